# Initial kernel scaffold; baseline (speedup 1.0000x reference)
import jax
import jax.numpy as jnp
from jax.experimental import pallas as pl


def kernel(train_matrix):
    raise NotImplementedError("write your pallas kernel here")


# speedup vs baseline: 2.5598x; 2.5598x over previous
import functools

import jax
import jax.numpy as jnp
from jax.experimental import pallas as pl
from jax.experimental.pallas import tpu as pltpu

L1_REG = 0.001
L2_REG = 0.01
TOPK = 100
EPS = 1e-7

_INTERPRET = False


def _prep_body(x_ref, a_ref, c_ref):
    x = x_ref[...]
    inp = jnp.clip(x, EPS, 1.0 - EPS)
    lg = jnp.log(inp)
    l1m = jnp.log1p(-inp)
    a_ref[...] = (lg - l1m).astype(jnp.bfloat16)

    @pl.when(pl.program_id(0) == 0)
    def _():
        c_ref[0, 0] = 0.0

    c_ref[0, 0] += jnp.sum(l1m)


def _prep(x, blk_rows=1024):
    u, n = x.shape
    grid = (u // blk_rows,)
    return pl.pallas_call(
        _prep_body,
        grid=grid,
        in_specs=[pl.BlockSpec((blk_rows, n), lambda i: (i, 0))],
        out_specs=[
            pl.BlockSpec((blk_rows, n), lambda i: (i, 0)),
            pl.BlockSpec(memory_space=pltpu.SMEM),
        ],
        out_shape=[
            jax.ShapeDtypeStruct((u, n), jnp.bfloat16),
            jax.ShapeDtypeStruct((1, 1), jnp.float32),
        ],
        interpret=_INTERPRET,
    )(x)


def _mm_body(lhs_ref, rhs_ref, out_ref, *, precision, nk):
    @pl.when(pl.program_id(1) == 0)
    def _():
        out_ref[...] = jnp.zeros_like(out_ref)

    out_ref[...] += jax.lax.dot_general(
        lhs_ref[...],
        rhs_ref[...],
        (((0,), (0,)), ((), ())),
        preferred_element_type=jnp.float32,
        precision=precision,
    )


def _matmul_t(lhs, rhs, precision, bi=1024, bk=1024):
    k, i = lhs.shape
    _, j = rhs.shape
    grid = (i // bi, k // bk)
    return pl.pallas_call(
        functools.partial(_mm_body, precision=precision, nk=k // bk),
        grid=grid,
        in_specs=[
            pl.BlockSpec((bk, bi), lambda ib, kb: (kb, ib)),
            pl.BlockSpec((bk, j), lambda ib, kb: (kb, 0)),
        ],
        out_specs=pl.BlockSpec((bi, j), lambda ib, kb: (ib, 0)),
        out_shape=jax.ShapeDtypeStruct((i, j), jnp.float32),
        interpret=_INTERPRET,
    )(lhs, rhs)


def _select_body(g_ref, m_ref, c_ref, out_ref, *, n_items, blk_cols, n_blocks,
                 n_users):
    g = g_ref[...]
    col = jax.lax.broadcasted_iota(jnp.int32, g.shape, 1) + (
        pl.program_id(0) * blk_cols)
    row = jax.lax.broadcasted_iota(jnp.int32, g.shape, 0)
    isdiag = row == col
    gi = jax.lax.bitcast_convert_type(g, jnp.int32)
    gi = jnp.where(isdiag, jnp.int32(-1), gi)
    diag = jnp.sum(jnp.where(isdiag, g, 0.0), axis=0, keepdims=True)

    def body(_, carry):
        lo, hi = carry
        mid = lo + ((hi - lo + 1) >> 1)
        cnt = jnp.sum((gi >= mid).astype(jnp.int32), axis=0, keepdims=True)
        ok = cnt >= TOPK
        return jnp.where(ok, mid, lo), jnp.where(ok, hi, mid - 1)

    lo = jnp.zeros((1, blk_cols), jnp.int32)
    hi = jnp.full((1, blk_cols), jnp.int32(0x7F7FFFFF))
    lo, hi = jax.lax.fori_loop(0, 31, body, (lo, hi))

    mask = gi >= lo
    w = jnp.maximum(g - L1_REG, 0.0) * (1.0 / (diag + L2_REG))
    s = jnp.sum(jnp.where(mask, w * m_ref[...], 0.0))

    @pl.when(pl.program_id(0) == 0)
    def _():
        out_ref[0, 0] = 0.0

    out_ref[0, 0] += s

    @pl.when(pl.program_id(0) == n_blocks - 1)
    def _():
        out_ref[0, 0] = (-(out_ref[0, 0] + c_ref[0, 0])) / n_users


def _select_reduce(g, m, c, n_users, blk_cols=256):
    n = g.shape[0]
    n_blocks = n // blk_cols
    return pl.pallas_call(
        functools.partial(_select_body, n_items=n, blk_cols=blk_cols,
                          n_blocks=n_blocks, n_users=n_users),
        grid=(n_blocks,),
        in_specs=[
            pl.BlockSpec((n, blk_cols), lambda i: (0, i)),
            pl.BlockSpec((n, blk_cols), lambda i: (0, i)),
            pl.BlockSpec(memory_space=pltpu.SMEM),
        ],
        out_specs=pl.BlockSpec(memory_space=pltpu.SMEM),
        out_shape=jax.ShapeDtypeStruct((1, 1), jnp.float32),
        interpret=_INTERPRET,
    )(g, m, c)


def kernel(train_matrix):
    x = train_matrix
    u, n = x.shape
    a_bf16, c = _prep(x)
    g = _matmul_t(x, x, jax.lax.Precision.HIGHEST)
    m = _matmul_t(x.astype(jnp.bfloat16), a_bf16, jax.lax.Precision.DEFAULT)
    loss = _select_reduce(g, m, c, u)
    return loss[0, 0]


# speedup vs baseline: 4.7822x; 1.8681x over previous
import functools

import jax
import jax.numpy as jnp
from jax.experimental import pallas as pl
from jax.experimental.pallas import tpu as pltpu

L1_REG = 0.001
L2_REG = 0.01
TOPK = 100
EPS = 1e-7

_INTERPRET = False


def _fused_body(lhs_ref, rhs_ref, out_ref, g_acc, m_acc, c_acc, s_acc, *,
                bj, n_users):
    j = pl.program_id(0)
    k = pl.program_id(1)
    nj = pl.num_programs(0)
    nk = pl.num_programs(1)

    @pl.when((j == 0) & (k == 0))
    def _():
        c_acc[0, 0] = 0.0
        s_acc[0, 0] = 0.0

    @pl.when(k == 0)
    def _():
        g_acc[...] = jnp.zeros_like(g_acc)
        m_acc[...] = jnp.zeros_like(m_acc)

    xl = lhs_ref[...]
    xr = rhs_ref[...]
    hi_l = xl.astype(jnp.bfloat16)
    lo_l = (xl - hi_l.astype(jnp.float32)).astype(jnp.bfloat16)
    hi_r = xr.astype(jnp.bfloat16)
    lo_r = (xr - hi_r.astype(jnp.float32)).astype(jnp.bfloat16)

    inp = jnp.clip(xr, EPS, 1.0 - EPS)
    l1m = jnp.log1p(-inp)
    a = (jnp.log(inp) - l1m).astype(jnp.bfloat16)
    c_acc[0, 0] += jnp.sum(l1m)

    dims = (((0,), (0,)), ((), ()))
    dot = functools.partial(jax.lax.dot_general, dimension_numbers=dims,
                            preferred_element_type=jnp.float32)
    g_acc[...] += dot(hi_l, hi_r)
    g_acc[...] += dot(hi_l, lo_r)
    g_acc[...] += dot(lo_l, hi_r)
    m_acc[...] += dot(hi_l, a)

    @pl.when(k == nk - 1)
    def _():
        g = g_acc[...]
        col = jax.lax.broadcasted_iota(jnp.int32, g.shape, 1) + j * bj
        row = jax.lax.broadcasted_iota(jnp.int32, g.shape, 0)
        isdiag = row == col
        gi = jax.lax.bitcast_convert_type(g, jnp.int32)
        gi = jnp.where(isdiag, jnp.int32(-1), gi)
        diag = jnp.sum(jnp.where(isdiag, g, 0.0), axis=0, keepdims=True)

        lo = jnp.zeros((1, bj), jnp.int32)
        hi = jnp.max(gi, axis=0, keepdims=True)

        def cond(carry):
            lo_, hi_ = carry
            return jnp.any(lo_ < hi_)

        def body(carry):
            lo_, hi_ = carry
            mid = lo_ + ((hi_ - lo_ + 1) >> 1)
            cnt = jnp.sum((gi >= mid).astype(jnp.int32), axis=0,
                          keepdims=True)
            ok = cnt >= TOPK
            return jnp.where(ok, mid, lo_), jnp.where(ok, hi_, mid - 1)

        lo, hi = jax.lax.while_loop(cond, body, (lo, hi))

        mask = gi >= lo
        w = jnp.maximum(g - L1_REG, 0.0) / (diag + L2_REG)
        s_acc[0, 0] += jnp.sum(jnp.where(mask, w * m_acc[...], 0.0))

        @pl.when(j == nj - 1)
        def _():
            out_ref[0, 0] = (-(s_acc[0, 0] + c_acc[0, 0])) / n_users


def _fused(x, bk=256, bj=1024):
    u, n = x.shape
    grid = (n // bj, u // bk)
    return pl.pallas_call(
        functools.partial(_fused_body, bj=bj, n_users=u),
        grid=grid,
        in_specs=[
            pl.BlockSpec((bk, n), lambda jb, kb: (kb, 0)),
            pl.BlockSpec((bk, bj), lambda jb, kb: (kb, jb)),
        ],
        out_specs=pl.BlockSpec(memory_space=pltpu.SMEM),
        out_shape=jax.ShapeDtypeStruct((1, 1), jnp.float32),
        scratch_shapes=[
            pltpu.VMEM((n, bj), jnp.float32),
            pltpu.VMEM((n, bj), jnp.float32),
            pltpu.SMEM((1, 1), jnp.float32),
            pltpu.SMEM((1, 1), jnp.float32),
        ],
        interpret=_INTERPRET,
    )(x, x)


def kernel(train_matrix):
    loss = _fused(train_matrix)
    return loss[0, 0]
